# baseline (speedup 1.0000x reference)
import functools

import jax
import jax.numpy as jnp
from jax import lax
from jax.experimental import pallas as pl
from jax.experimental.pallas import tpu as pltpu
from jax.experimental.pallas import tpu_sc as plsc

D_MODEL = 32
_NC = 2
_NS = 16
_NW = _NC * _NS
_CHUNK = 128


@functools.cache
def _make_gather(B: int):
    assert B % (_NW * _CHUNK) == 0
    b_per_w = B // _NW
    n_chunks = b_per_w // _CHUNK
    mesh = plsc.VectorSubcoreMesh(core_axis_name="c", subcore_axis_name="s")

    @functools.partial(
        pl.kernel,
        mesh=mesh,
        out_type=jax.ShapeDtypeStruct((B, D_MODEL), jnp.float32),
        scratch_types=[
            pltpu.VMEM((_CHUNK,), jnp.int32),
            pltpu.VMEM((_CHUNK, D_MODEL), jnp.float32),
            pltpu.SemaphoreType.DMA,
        ],
        compiler_params=pltpu.CompilerParams(use_tc_tiling_on_sc=False),
    )
    def gather(table_hbm, idx_hbm, out_hbm, idx_v, rows_v, sem):
        wid = lax.axis_index("s") * _NC + lax.axis_index("c")
        base = wid * b_per_w

        def body(i, carry):
            off = base + i * _CHUNK
            pltpu.sync_copy(idx_hbm.at[pl.ds(off, _CHUNK)], idx_v)
            pltpu.async_copy(table_hbm.at[idx_v], rows_v, sem).wait()
            pltpu.sync_copy(rows_v, out_hbm.at[pl.ds(off, _CHUNK)])
            return carry

        lax.fori_loop(0, n_chunks, body, 0)

    return gather


def kernel(x, W):
    B = x.shape[0] * x.shape[1]
    idx = x.reshape(B).astype(jnp.int32)
    out = _make_gather(B)(W, idx)
    return out.reshape(x.shape[0], x.shape[1], D_MODEL)


# speedup vs baseline: 1.1386x; 1.1386x over previous
import functools

import jax
import jax.numpy as jnp
from jax import lax
from jax.experimental import pallas as pl
from jax.experimental.pallas import tpu as pltpu
from jax.experimental.pallas import tpu_sc as plsc

D_MODEL = 32
_NC = 2
_NS = 16
_NW = _NC * _NS
_CHUNK = 128
_NBUF = 8


@functools.cache
def _make_gather(B: int):
    assert B % (_NW * _CHUNK * _NBUF) == 0
    b_per_w = B // _NW
    n_chunks = b_per_w // _CHUNK
    n_groups = n_chunks // _NBUF
    mesh = plsc.VectorSubcoreMesh(core_axis_name="c", subcore_axis_name="s")

    @functools.partial(
        pl.kernel,
        mesh=mesh,
        out_type=jax.ShapeDtypeStruct((B, D_MODEL), jnp.float32),
        scratch_types=[
            pltpu.VMEM((b_per_w,), jnp.int32),
            [pltpu.VMEM((_CHUNK, D_MODEL), jnp.float32) for _ in range(_NBUF)],
            [pltpu.SemaphoreType.DMA for _ in range(_NBUF)],
            [pltpu.SemaphoreType.DMA for _ in range(_NBUF)],
        ],
        compiler_params=pltpu.CompilerParams(use_tc_tiling_on_sc=False),
    )
    def gather(table_hbm, idx_hbm, out_hbm, idx_all, rows, gsem, wsem):
        wid = lax.axis_index("s") * _NC + lax.axis_index("c")
        base = wid * b_per_w
        pltpu.sync_copy(idx_hbm.at[pl.ds(base, b_per_w)], idx_all)

        def start_gather(chunk, b):
            pltpu.async_copy(
                table_hbm.at[idx_all.at[pl.ds(chunk * _CHUNK, _CHUNK)]],
                rows[b], gsem[b])

        def wait_gather(b):
            pltpu.make_async_copy(
                table_hbm.at[pl.ds(0, _CHUNK)], rows[b], gsem[b]).wait()

        def start_wb(chunk, b):
            pltpu.async_copy(
                rows[b], out_hbm.at[pl.ds(base + chunk * _CHUNK, _CHUNK)],
                wsem[b])

        def wait_wb(b):
            pltpu.make_async_copy(
                rows[b], out_hbm.at[pl.ds(base, _CHUNK)], wsem[b]).wait()

        for b in range(_NBUF):
            start_gather(b, b)

        def body(j, carry):
            for b in range(_NBUF):
                wait_gather(b)
                start_wb(j * _NBUF + b, b)
            for b in range(_NBUF):
                wait_wb(b)
                start_gather((j + 1) * _NBUF + b, b)
            return carry

        lax.fori_loop(0, n_groups - 1, body, 0)

        last = (n_groups - 1) * _NBUF
        for b in range(_NBUF):
            wait_gather(b)
            start_wb(last + b, b)
        for b in range(_NBUF):
            wait_wb(b)

    return gather


def kernel(x, W):
    B = x.shape[0] * x.shape[1]
    idx = x.reshape(B).astype(jnp.int32)
    out = _make_gather(B)(W, idx)
    return out.reshape(x.shape[0], x.shape[1], D_MODEL)


# speedup vs baseline: 1.4178x; 1.2452x over previous
import functools

import jax
import jax.numpy as jnp
from jax import lax
from jax.experimental import pallas as pl
from jax.experimental.pallas import tpu as pltpu
from jax.experimental.pallas import tpu_sc as plsc

D_MODEL = 32
_NC = 2
_NS = 16
_NW = _NC * _NS
_CHUNK = 128
_NBUF = 2
_L = 16


@functools.cache
def _make_gather(N_I: int, N_J: int):
    i_per_w = N_I // _NW
    n_chunks_j = i_per_w // _CHUNK
    n_chunks = N_J * n_chunks_j
    n_groups = n_chunks // _NBUF
    mesh = plsc.VectorSubcoreMesh(core_axis_name="c", subcore_axis_name="s")

    @functools.partial(
        pl.kernel,
        mesh=mesh,
        out_type=jax.ShapeDtypeStruct((N_J, D_MODEL, N_I), jnp.float32),
        scratch_types=[
            pltpu.VMEM((N_J, i_per_w), jnp.int32),
            [pltpu.VMEM((_CHUNK, D_MODEL), jnp.float32) for _ in range(_NBUF)],
            [pltpu.VMEM((D_MODEL, _CHUNK), jnp.float32) for _ in range(_NBUF)],
            [pltpu.SemaphoreType.DMA for _ in range(_NBUF)],
            [pltpu.SemaphoreType.DMA for _ in range(_NBUF)],
        ],
        compiler_params=pltpu.CompilerParams(
            use_tc_tiling_on_sc=False, needs_layout_passes=False),
    )
    def gather(table_hbm, idxt_hbm, out_hbm, idx_v, rows, trans, gsem, wsem):
        wid = lax.axis_index("s") * _NC + lax.axis_index("c")
        i0 = wid * i_per_w
        pltpu.sync_copy(idxt_hbm.at[:, pl.ds(i0, i_per_w)], idx_v)

        def start_gather(c, b):
            j = c // n_chunks_j
            col = (c % n_chunks_j) * _CHUNK
            pltpu.async_copy(
                table_hbm.at[idx_v.at[j, pl.ds(col, _CHUNK)]],
                rows[b], gsem[b])

        def wait_gather(b):
            pltpu.make_async_copy(
                table_hbm.at[pl.ds(0, _CHUNK)], rows[b], gsem[b]).wait()

        def start_wb(c, b):
            j = c // n_chunks_j
            col = i0 + (c % n_chunks_j) * _CHUNK
            pltpu.async_copy(
                trans[b], out_hbm.at[j, :, pl.ds(col, _CHUNK)], wsem[b])

        def wait_wb(b):
            pltpu.make_async_copy(
                trans[b], out_hbm.at[0, :, pl.ds(0, _CHUNK)], wsem[b]).wait()

        def transpose(b):
            iota = lax.iota(jnp.int32, _L)
            for d in range(D_MODEL):
                col = jnp.full((_L,), d, jnp.int32)
                for l0 in range(0, _CHUNK, _L):
                    row = iota + l0
                    v = plsc.load_gather(rows[b], [row, col])
                    trans[b][d, pl.ds(l0, _L)] = v

        for b in range(_NBUF):
            start_gather(b, b)

        def body(g, carry):
            for b in range(_NBUF):
                c = g * _NBUF + b
                wait_gather(b)

                @pl.when(g != 0)
                def _():
                    wait_wb(b)

                transpose(b)
                start_wb(c, b)

                @pl.when(g < n_groups - 1)
                def _():
                    start_gather(c + _NBUF, b)

            return carry

        lax.fori_loop(0, n_groups, body, 0)
        for b in range(_NBUF):
            wait_wb(b)

    return gather


def kernel(x, W):
    n_b, n_h = x.shape
    xt = x.astype(jnp.int32).T
    out_t = _make_gather(n_b, n_h)(W, xt)
    return jnp.transpose(out_t, (2, 0, 1))


# speedup vs baseline: 1.8524x; 1.3065x over previous
import functools

import jax
import jax.numpy as jnp
from jax import lax
from jax.experimental import pallas as pl
from jax.experimental.pallas import tpu as pltpu
from jax.experimental.pallas import tpu_sc as plsc

D_MODEL = 32
_NC = 2
_NS = 16
_NW = _NC * _NS
_CHUNK = 128
_NBUF = 2
_L = 16


@functools.cache
def _make_gather(N_I: int, N_J: int):
    i_per_w = N_I // _NW
    n_chunks_j = i_per_w // _CHUNK
    n_chunks = N_J * n_chunks_j
    n_groups = n_chunks // _NBUF
    mesh = plsc.VectorSubcoreMesh(core_axis_name="c", subcore_axis_name="s")

    @functools.partial(
        pl.kernel,
        mesh=mesh,
        out_type=jax.ShapeDtypeStruct((N_J, D_MODEL, N_I), jnp.float32),
        scratch_types=[
            pltpu.VMEM((N_J, i_per_w), jnp.int32),
            [pltpu.VMEM((_CHUNK, D_MODEL), jnp.float32) for _ in range(_NBUF)],
            [pltpu.VMEM((D_MODEL, _CHUNK), jnp.float32) for _ in range(_NBUF)],
            [pltpu.SemaphoreType.DMA for _ in range(_NBUF)],
            [pltpu.SemaphoreType.DMA for _ in range(_NBUF)],
        ],
        compiler_params=pltpu.CompilerParams(
            use_tc_tiling_on_sc=False, needs_layout_passes=False),
    )
    def gather(table_hbm, idxt_hbm, out_hbm, idx_v, rows, trans, gsem, wsem):
        wid = lax.axis_index("s") * _NC + lax.axis_index("c")
        i0 = wid * i_per_w
        pltpu.sync_copy(idxt_hbm.at[:, pl.ds(i0, i_per_w)], idx_v)

        def start_gather(c, b):
            j = c // n_chunks_j
            col = (c % n_chunks_j) * _CHUNK
            pltpu.async_copy(
                table_hbm.at[idx_v.at[j, pl.ds(col, _CHUNK)]],
                rows[b], gsem[b])

        def wait_gather(b):
            pltpu.make_async_copy(
                table_hbm.at[pl.ds(0, _CHUNK)], rows[b], gsem[b]).wait()

        def start_wb(c, b):
            j = c // n_chunks_j
            col = i0 + (c % n_chunks_j) * _CHUNK
            pltpu.async_copy(
                trans[b], out_hbm.at[j, :, pl.ds(col, _CHUNK)], wsem[b])

        def wait_wb(b):
            pltpu.make_async_copy(
                trans[b], out_hbm.at[0, :, pl.ds(0, _CHUNK)], wsem[b]).wait()

        def transpose(b):
            iota = lax.iota(jnp.int32, _L)
            for l0 in range(0, _CHUNK, _L):
                row = iota + l0
                for d0 in range(0, D_MODEL, _L):
                    for k in range(_L):
                        col = ((iota + k) & (_L - 1)) + d0
                        v = plsc.load_gather(rows[b], [row, col])
                        plsc.store_scatter(trans[b], [col, row], v)

        for b in range(_NBUF):
            start_gather(b, b)

        def body(g, carry):
            for b in range(_NBUF):
                c = g * _NBUF + b
                wait_gather(b)

                @pl.when(g != 0)
                def _():
                    wait_wb(b)

                transpose(b)
                start_wb(c, b)

                @pl.when(g < n_groups - 1)
                def _():
                    start_gather(c + _NBUF, b)

            return carry

        lax.fori_loop(0, n_groups, body, 0)
        for b in range(_NBUF):
            wait_wb(b)

    return gather


def kernel(x, W):
    n_b, n_h = x.shape
    xt = x.astype(jnp.int32).T
    out_t = _make_gather(n_b, n_h)(W, xt)
    return jnp.transpose(out_t, (2, 0, 1))


# speedup vs baseline: 2.2952x; 1.2391x over previous
import functools

import jax
import jax.numpy as jnp
from jax import lax
from jax.experimental import pallas as pl
from jax.experimental.pallas import tpu as pltpu
from jax.experimental.pallas import tpu_sc as plsc

D_MODEL = 32
_NC = 2
_NS = 16
_NW = _NC * _NS
_CHUNK = 128
_NBUF = 4
_L = 16


@functools.cache
def _make_gather(N_I: int, N_J: int):
    i_per_w = N_I // _NW
    n_chunks_j = i_per_w // _CHUNK
    n_chunks = N_J * n_chunks_j
    n_groups = n_chunks // _NBUF
    mesh = plsc.VectorSubcoreMesh(core_axis_name="c", subcore_axis_name="s")

    @functools.partial(
        pl.kernel,
        mesh=mesh,
        out_type=jax.ShapeDtypeStruct((N_J, D_MODEL, N_I), jnp.float32),
        scratch_types=[
            pltpu.VMEM((N_J, i_per_w), jnp.int32),
            [pltpu.VMEM((_CHUNK, D_MODEL), jnp.float32) for _ in range(_NBUF)],
            [pltpu.VMEM((D_MODEL, _CHUNK), jnp.float32) for _ in range(_NBUF)],
            [pltpu.SemaphoreType.DMA for _ in range(_NBUF)],
            [pltpu.SemaphoreType.DMA for _ in range(_NBUF)],
        ],
        compiler_params=pltpu.CompilerParams(
            use_tc_tiling_on_sc=False, needs_layout_passes=False),
    )
    def gather(table_hbm, idxt_hbm, out_hbm, idx_v, rows, trans, gsem, wsem):
        wid = lax.axis_index("s") * _NC + lax.axis_index("c")
        i0 = wid * i_per_w
        pltpu.sync_copy(idxt_hbm.at[:, pl.ds(i0, i_per_w)], idx_v)

        def start_gather(c, b):
            j = c // n_chunks_j
            col = (c % n_chunks_j) * _CHUNK
            pltpu.async_copy(
                table_hbm.at[idx_v.at[j, pl.ds(col, _CHUNK)]],
                rows[b], gsem[b])

        def wait_gather(b):
            pltpu.make_async_copy(
                table_hbm.at[pl.ds(0, _CHUNK)], rows[b], gsem[b]).wait()

        def start_wb(c, b):
            j = c // n_chunks_j
            col = i0 + (c % n_chunks_j) * _CHUNK
            pltpu.async_copy(
                trans[b], out_hbm.at[j, :, pl.ds(col, _CHUNK)], wsem[b])

        def wait_wb(b):
            pltpu.make_async_copy(
                trans[b], out_hbm.at[0, :, pl.ds(0, _CHUNK)], wsem[b]).wait()

        def transpose(b):
            iota = lax.iota(jnp.int32, _L)

            def kbody(k, carry):
                colbase = (iota + k) & (_L - 1)
                for l0 in range(0, _CHUNK, _L):
                    row = iota + l0
                    for d0 in range(0, D_MODEL, _L):
                        col = colbase + d0
                        v = plsc.load_gather(rows[b], [row, col])
                        plsc.store_scatter(trans[b], [col, row], v)
                return carry

            lax.fori_loop(0, _L, kbody, 0)

        for b in range(_NBUF):
            start_gather(b, b)

        def body(g, carry):
            for b in range(_NBUF):
                c = g * _NBUF + b
                wait_gather(b)

                @pl.when(g != 0)
                def _():
                    wait_wb(b)

                transpose(b)
                start_wb(c, b)

                @pl.when(g < n_groups - 1)
                def _():
                    start_gather(c + _NBUF, b)

            return carry

        lax.fori_loop(0, n_groups, body, 0)
        for b in range(_NBUF):
            wait_wb(b)

    return gather


def kernel(x, W):
    n_b, n_h = x.shape
    xt = x.astype(jnp.int32).T
    out_t = _make_gather(n_b, n_h)(W, xt)
    return jnp.transpose(out_t, (2, 0, 1))
